# baseline (speedup 1.0000x reference)
import functools

import jax
import jax.numpy as jnp
from jax import lax
from jax.experimental import pallas as pl
from jax.experimental.pallas import tpu as pltpu
from jax.experimental.pallas import tpu_sc as plsc

N = 50000
E = 800000
H = 64

NC = 2
NS = 16
NW = NC * NS
NPC = N // NC
SROWS = 25600
DUMP = SROWS - 1
ZROWS = SROWS // NS
EPAD = 819200
ET = EPAD // NW
C = 1024
J = C // 128
NCHUNK = ET // C
OCH = 1000
NOCH = NPC // OCH

CL = 256
JL = CL // 128
ET2 = EPAD // NS
NCHUNKL = ET2 // CL
HP = 128
RNG = 10240
NRANGE = 5
DUMPL = RNG
SROWSL = RNG + 8
OCHL = 40
NCH_R = RNG // OCHL
NCH_LAST = (N - 4 * RNG) // OCHL

BN = 2000
GRID = N // BN

_mesh = functools.partial(plsc.VectorSubcoreMesh,
                          core_axis_name="c", subcore_axis_name="s")


def _relu(v):
    return jnp.maximum(v, 0.0)


def _compute_lidx(dst_v, lidx, node_base, n):
    for t in range(n // 16):
        d = dst_v[pl.ds(t * 16, 16)]
        l = d - node_base
        valid = jnp.logical_and(l >= 0, l < NPC)
        lidx[t // 8, pl.ds((t % 8) * 16, 16)] = jnp.where(valid, l, DUMP)


def _sc_deg_body(dstp_hbm, deg_hbm, dst_v, lidx, ones_v, zbuf, deg_sh):
    c = lax.axis_index("c")
    s = lax.axis_index("s")
    node_base = c * NPC
    for t in range(8):
        ones_v[pl.ds(t * 16, 16)] = jnp.full((16,), 1.0, jnp.float32)
    for t in range(ZROWS // 16):
        zbuf[pl.ds(t * 16, 16)] = jnp.zeros((16,), jnp.float32)
    pltpu.sync_copy(zbuf, deg_sh.at[pl.ds(s * ZROWS, ZROWS)])
    plsc.subcore_barrier()

    def chunk(ci, _):
        off = s * ET2 + ci * C
        pltpu.sync_copy(dstp_hbm.at[pl.ds(off, C)], dst_v)
        _compute_lidx(dst_v, lidx, node_base, C)
        for j in range(J):
            pltpu.sync_copy(ones_v, deg_sh.at[lidx.at[j]], add=True)
        return _

    lax.fori_loop(0, ET2 // C, chunk, None)
    plsc.subcore_barrier()
    for k in range(2):
        ch = s + NS * k

        @pl.when(ch < NOCH)
        def _():
            pltpu.sync_copy(deg_sh.at[pl.ds(ch * OCH, OCH)],
                            zbuf.at[pl.ds(0, OCH)])
            pltpu.sync_copy(zbuf.at[pl.ds(0, OCH)],
                            deg_hbm.at[pl.ds(node_base + ch * OCH, OCH)])


def _sc_deg(dstp):
    return pl.kernel(
        _sc_deg_body,
        out_type=jax.ShapeDtypeStruct((N,), jnp.float32),
        mesh=_mesh(),
        scratch_types=[
            pltpu.VMEM((C,), jnp.int32),
            pltpu.VMEM((J, 128), jnp.int32),
            pltpu.VMEM((128,), jnp.float32),
            pltpu.VMEM((ZROWS,), jnp.float32),
            pltpu.VMEM_SHARED((SROWS,), jnp.float32),
        ],
    )(dstp)


def _lidx_range(dst_v, lidx, rb):
    for t in range(CL // 16):
        d = dst_v[pl.ds(t * 16, 16)]
        l = d - rb
        valid = jnp.logical_and(l >= 0, l < RNG)
        lidx[t // 8, pl.ds((t % 8) * 16, 16)] = jnp.where(valid, l, DUMPL)


def _sc_layer_body(g_hbm, srcp_hbm, dstp_hbm, out_hbm,
                   src_v, dst_v, lidx, rows_v, obuf, acc_sh, sem):
    c = lax.axis_index("c")
    s = lax.axis_index("s")

    for p in range(3):
        r = 2 * p + c

        @pl.when(r < NRANGE)
        def _(r=r):
            rb = r * RNG
            nch = jnp.where(r == NRANGE - 1, NCH_LAST, NCH_R)
            for q in range(OCHL):
                for t in range(HP // 16):
                    obuf[q, pl.ds(t * 16, 16)] = jnp.zeros((16,), jnp.float32)
            for k in range(NCH_R // NS):
                pltpu.sync_copy(
                    obuf, acc_sh.at[pl.ds((s + NS * k) * OCHL, OCHL)])
            plsc.subcore_barrier()

            def chunk(ci, _):
                off = s * ET2 + ci * CL
                pltpu.sync_copy(srcp_hbm.at[pl.ds(off, CL)], src_v)
                pltpu.sync_copy(dstp_hbm.at[pl.ds(off, CL)], dst_v)
                _lidx_range(dst_v, lidx, rb)
                cps = []
                for j in range(JL):
                    cps.append(pltpu.async_copy(
                        g_hbm.at[src_v.at[pl.ds(j * 128, 128)]],
                        rows_v.at[j], sem))
                for cp in cps:
                    cp.wait()
                for j in range(JL):
                    pltpu.sync_copy(rows_v.at[j], acc_sh.at[lidx.at[j]],
                                    add=True)
                return _

            lax.fori_loop(0, NCHUNKL, chunk, None)
            plsc.subcore_barrier()
            for k in range(NCH_R // NS):
                ch = s + NS * k

                @pl.when(ch < nch)
                def _(ch=ch):
                    pltpu.sync_copy(acc_sh.at[pl.ds(ch * OCHL, OCHL)], obuf)
                    pltpu.sync_copy(
                        obuf, out_hbm.at[pl.ds(rb + ch * OCHL, OCHL)])
            plsc.subcore_barrier()


def _sc_layer(g, srcp, dstp):
    return pl.kernel(
        _sc_layer_body,
        out_type=jax.ShapeDtypeStruct((NRANGE * RNG, HP), jnp.float32),
        mesh=_mesh(),
        scratch_types=[
            pltpu.VMEM((CL,), jnp.int32),
            pltpu.VMEM((CL,), jnp.int32),
            pltpu.VMEM((JL, 128), jnp.int32),
            pltpu.VMEM((JL, 128, HP), jnp.float32),
            pltpu.VMEM((OCHL, HP), jnp.float32),
            pltpu.VMEM_SHARED((SROWSL, HP), jnp.float32),
            pltpu.SemaphoreType.DMA,
        ],
    )(g, srcp, dstp)


def _enc_body(x_ref, deg_ref, w1_ref, b1_ref, w2_ref, b2_ref, w0_ref,
              h_ref, g_ref, dis_ref):
    h = _relu(x_ref[...] @ w1_ref[...] + b1_ref[...][None, :])
    h = _relu(h @ w2_ref[...] + b2_ref[...][None, :])
    dis = lax.rsqrt(deg_ref[...] + 1.0)
    h_ref[...] = h
    g_ref[...] = (h @ w0_ref[...]) * dis
    dis_ref[...] = dis


def _tc_encoder(x, deg, ne_w1, ne_b1, ne_w2, ne_b2, w0):
    return pl.pallas_call(
        _enc_body,
        grid=(GRID,),
        in_specs=[
            pl.BlockSpec((BN, 4), lambda i: (i, 0)),
            pl.BlockSpec((BN, 1), lambda i: (i, 0)),
            pl.BlockSpec((4, H), lambda i: (0, 0)),
            pl.BlockSpec((H,), lambda i: (0,)),
            pl.BlockSpec((H, H), lambda i: (0, 0)),
            pl.BlockSpec((H,), lambda i: (0,)),
            pl.BlockSpec((H, HP), lambda i: (0, 0)),
        ],
        out_specs=[
            pl.BlockSpec((BN, H), lambda i: (i, 0)),
            pl.BlockSpec((BN, HP), lambda i: (i, 0)),
            pl.BlockSpec((BN, 1), lambda i: (i, 0)),
        ],
        out_shape=[
            jax.ShapeDtypeStruct((N, H), jnp.float32),
            jax.ShapeDtypeStruct((N, HP), jnp.float32),
            jax.ShapeDtypeStruct((N, 1), jnp.float32),
        ],
    )(x, deg, ne_w1, ne_b1, ne_w2, ne_b2, w0)


def _upd_body(h_ref, g_ref, acc_ref, dis_ref, b_ref, wn_ref,
              h2_ref, g2_ref):
    dis = dis_ref[...]
    acc = acc_ref[...][:, :H] + g_ref[...][:, :H]
    hn = h_ref[...] + _relu(dis * acc + b_ref[...][None, :])
    h2_ref[...] = hn
    g2_ref[...] = (hn @ wn_ref[...]) * dis


def _tc_update(h, g, acc, dis, b, wn):
    return pl.pallas_call(
        _upd_body,
        grid=(GRID,),
        in_specs=[
            pl.BlockSpec((BN, H), lambda i: (i, 0)),
            pl.BlockSpec((BN, HP), lambda i: (i, 0)),
            pl.BlockSpec((BN, HP), lambda i: (i, 0)),
            pl.BlockSpec((BN, 1), lambda i: (i, 0)),
            pl.BlockSpec((H,), lambda i: (0,)),
            pl.BlockSpec((H, HP), lambda i: (0, 0)),
        ],
        out_specs=[
            pl.BlockSpec((BN, H), lambda i: (i, 0)),
            pl.BlockSpec((BN, HP), lambda i: (i, 0)),
        ],
        out_shape=[
            jax.ShapeDtypeStruct((N, H), jnp.float32),
            jax.ShapeDtypeStruct((N, HP), jnp.float32),
        ],
    )(h, g, acc, dis, b, wn)


def _fin_body(h_ref, g_ref, acc_ref, dis_ref, b_ref, h2_ref):
    acc = acc_ref[...][:, :H] + g_ref[...][:, :H]
    h2_ref[...] = h_ref[...] + _relu(
        dis_ref[...] * acc + b_ref[...][None, :])


def _tc_final(h, g, acc, dis, b):
    return pl.pallas_call(
        _fin_body,
        grid=(GRID,),
        in_specs=[
            pl.BlockSpec((BN, H), lambda i: (i, 0)),
            pl.BlockSpec((BN, HP), lambda i: (i, 0)),
            pl.BlockSpec((BN, HP), lambda i: (i, 0)),
            pl.BlockSpec((BN, 1), lambda i: (i, 0)),
            pl.BlockSpec((H,), lambda i: (0,)),
        ],
        out_specs=[pl.BlockSpec((BN, H), lambda i: (i, 0))],
        out_shape=[jax.ShapeDtypeStruct((N, H), jnp.float32)],
    )(h, g, acc, dis, b)[0]


NPADX = 51200


def _sel_body(xc_ref, o_ref):
    ii = (lax.broadcasted_iota(jnp.int32, (NPADX // 128, 128), 0) * 128
          + lax.broadcasted_iota(jnp.int32, (NPADX // 128, 128), 1))
    sm = xc_ref[0] == 1.0
    tm = xc_ref[1] == 1.0
    s_first = jnp.min(jnp.where(sm, ii, NPADX))
    t_first = jnp.min(jnp.where(tm, ii, NPADX))
    has = jnp.logical_and(s_first < NPADX, t_first < NPADX)
    s_idx = jnp.where(has, s_first, 0)
    t_idx = jnp.where(has, t_first, N - 1)
    o_ref[...] = jnp.concatenate(
        [s_idx.reshape(1, 1), t_idx.reshape(1, 1)], axis=1)


def _tc_select(xc):
    return pl.pallas_call(
        _sel_body,
        out_shape=jax.ShapeDtypeStruct((1, 2), jnp.int32),
    )(xc)


def _gather_body(idx_ref, h_ref, o_ref):
    i = pl.program_id(0)
    rows = (lax.broadcasted_iota(jnp.int32, (BN, 1), 0) + i * BN)
    hv = h_ref[...]
    cs = jnp.sum(jnp.where(rows == idx_ref[0, 0], hv, 0.0), axis=0,
                 keepdims=True)
    ct = jnp.sum(jnp.where(rows == idx_ref[0, 1], hv, 0.0), axis=0,
                 keepdims=True)

    @pl.when(i == 0)
    def _():
        o_ref[...] = jnp.zeros((2, H), jnp.float32)

    o_ref[0:1, :] += cs
    o_ref[1:2, :] += ct


def _tc_gather(idx, h3):
    return pl.pallas_call(
        _gather_body,
        grid=(GRID,),
        in_specs=[
            pl.BlockSpec(memory_space=pltpu.SMEM),
            pl.BlockSpec((BN, H), lambda i: (i, 0)),
        ],
        out_specs=pl.BlockSpec((2, H), lambda i: (0, 0)),
        out_shape=jax.ShapeDtypeStruct((2, H), jnp.float32),
    )(idx, h3)


def _head_body(hs_ref, w1_ref, b1_ref, w2_ref, b2_ref, w3_ref, b3_ref, o_ref):
    o = _relu(hs_ref[0:1, :] @ w1_ref[0:H, :]
              + hs_ref[1:2, :] @ w1_ref[H:2 * H, :] + b1_ref[...][None, :])
    o = _relu(o @ w2_ref[...] + b2_ref[...][None, :])
    o_ref[...] = o @ w3_ref[...] + b3_ref[...][None, :]


def _tc_head(hs, w1, b1, w2, b2, w3, b3):
    return pl.pallas_call(
        _head_body,
        out_shape=jax.ShapeDtypeStruct((1, 1), jnp.float32),
    )(hs, w1, b1, w2, b2, w3, b3)


def kernel(x, edge_index, edge_attr, ne_w1, ne_b1, ne_w2, ne_b2, ee_w1, ee_b1,
           ee_w2, ee_b2, conv_ws, conv_bs, out_w1, out_b1, out_w2, out_b2,
           out_w3, out_b3):
    src = edge_index[0]
    dst = edge_index[1]
    pad = EPAD - E
    srcp = jnp.concatenate([src, jnp.zeros((pad,), jnp.int32)])
    dstp = jnp.concatenate([dst, jnp.full((pad,), N, jnp.int32)])
    xc = jnp.pad(x[:, 2:4].T, ((0, 0), (0, NPADX - N))).reshape(
        2, NPADX // 128, 128)
    wps = jnp.pad(conv_ws, ((0, 0), (0, 0), (0, HP - H)))

    deg = _sc_deg(dstp)
    h, g, dis = _tc_encoder(x, deg.reshape(N, 1), ne_w1, ne_b1, ne_w2, ne_b2,
                            wps[0])
    for i in range(3):
        acc = _sc_layer(g, srcp, dstp)[:N]
        if i < 2:
            h, g = _tc_update(h, g, acc, dis, conv_bs[i], wps[i + 1])
        else:
            h = _tc_final(h, g, acc, dis, conv_bs[i])
    idx = _tc_select(xc)
    hs = _tc_gather(idx, h)
    o = _tc_head(hs, out_w1, out_b1, out_w2, out_b2, out_w3, out_b3)
    return o.reshape(1)
